# baseline (device time: 35531 ns/iter reference)
import jax
import jax.numpy as jnp
from jax import lax
from jax.experimental import pallas as pl
from jax.experimental.pallas import tpu as pltpu

N_DEV = 4
B, Sq, Hq, Hkv, Dh = 2, 256, 8, 2, 64
G = Hq // Hkv
SCALE = 0.125


def kernel(x, Wq, Wo, K_ext, V_ext):
    c = K_ext.shape[1]

    def body(x_ref, wq_ref, wo_ref, k_ref, v_ref, out_ref,
             kbuf, vbuf, k_send, k_recv, v_send, v_recv):
        my = lax.axis_index("i")

        barrier = pltpu.get_barrier_semaphore()
        for d in range(1, N_DEV):
            pl.semaphore_signal(barrier, inc=1, device_id=((my + d) % N_DEV,),
                                device_id_type=pl.DeviceIdType.MESH)
        pl.semaphore_wait(barrier, N_DEV - 1)

        for b in range(B):
            kb = k_ref[b].astype(jnp.bfloat16)
            vb = v_ref[b].astype(jnp.bfloat16)
            for kvh in range(Hkv):
                kbuf[0, b, kvh] = jnp.transpose(kb[:, kvh, :])
                vbuf[0, b, kvh] = vb[:, kvh, :]

        krd, vrd = {}, {}
        for d in range(1, N_DEV):
            tgt = (my + d) % N_DEV
            krd[d] = pltpu.make_async_remote_copy(
                src_ref=kbuf.at[0], dst_ref=kbuf.at[N_DEV - d],
                send_sem=k_send.at[d - 1], recv_sem=k_recv.at[N_DEV - 1 - d],
                device_id=(tgt,), device_id_type=pl.DeviceIdType.MESH)
            vrd[d] = pltpu.make_async_remote_copy(
                src_ref=vbuf.at[0], dst_ref=vbuf.at[N_DEV - d],
                send_sem=v_send.at[d - 1], recv_sem=v_recv.at[N_DEV - 1 - d],
                device_id=(tgt,), device_id_type=pl.DeviceIdType.MESH)
            krd[d].start()
            vrd[d].start()

        qg = []
        for b in range(B):
            qb = jnp.dot(x_ref[b].astype(jnp.bfloat16),
                         wq_ref[...].astype(jnp.bfloat16),
                         preferred_element_type=jnp.float32)
            qbh = (qb * SCALE).astype(jnp.bfloat16)
            qg.append([
                jnp.concatenate(
                    [qbh[:, (kvh * G + g) * Dh:(kvh * G + g + 1) * Dh]
                     for g in range(G)], axis=0)
                for kvh in range(Hkv)
            ])

        l = [[jnp.zeros((G * Sq, 1), jnp.float32)
              for _ in range(Hkv)] for _ in range(B)]
        acc = [[jnp.zeros((G * Sq, Dh), jnp.float32)
                for _ in range(Hkv)] for _ in range(B)]

        def fold_slot(slot):
            for b in range(B):
                for kvh in range(Hkv):
                    kc = kbuf[slot, b, kvh]
                    vc = vbuf[slot, b, kvh]
                    s = jnp.dot(qg[b][kvh], kc,
                                preferred_element_type=jnp.float32)
                    p = jnp.exp(s)
                    l[b][kvh] = l[b][kvh] + jnp.sum(
                        p, axis=-1, keepdims=True)
                    acc[b][kvh] = acc[b][kvh] + jnp.dot(
                        p.astype(jnp.bfloat16), vc,
                        preferred_element_type=jnp.float32)

        fold_slot(0)
        for slot, d in ((3, 1), (1, 3), (2, 2)):
            krd[d].wait_recv()
            vrd[d].wait_recv()
            fold_slot(slot)

        wo_b = wo_ref[...].astype(jnp.bfloat16)
        for b in range(B):
            heads = []
            for h in range(Hq):
                kvh, g = h // G, h % G
                o = (acc[b][kvh][g * Sq:(g + 1) * Sq, :]
                     / l[b][kvh][g * Sq:(g + 1) * Sq, :])
                heads.append(o)
            ob = jnp.concatenate(heads, axis=1)
            out_ref[b] = jnp.dot(ob.astype(jnp.bfloat16), wo_b,
                                 preferred_element_type=jnp.float32)

        for d in range(1, N_DEV):
            krd[d].wait_send()
            vrd[d].wait_send()

    return pl.pallas_call(
        body,
        out_shape=jax.ShapeDtypeStruct((B, Sq, Wo.shape[1]), jnp.float32),
        in_specs=[pl.BlockSpec(memory_space=pltpu.VMEM)] * 5,
        out_specs=pl.BlockSpec(memory_space=pltpu.VMEM),
        scratch_shapes=[
            pltpu.VMEM((N_DEV, B, Hkv, Dh, c), jnp.bfloat16),
            pltpu.VMEM((N_DEV, B, Hkv, c, Dh), jnp.bfloat16),
            pltpu.SemaphoreType.DMA((N_DEV - 1,)),
            pltpu.SemaphoreType.DMA((N_DEV - 1,)),
            pltpu.SemaphoreType.DMA((N_DEV - 1,)),
            pltpu.SemaphoreType.DMA((N_DEV - 1,)),
        ],
        compiler_params=pltpu.CompilerParams(collective_id=0),
    )(x, Wq, Wo, K_ext, V_ext)


# device time: 32154 ns/iter; 1.1050x vs baseline; 1.1050x over previous
import jax
import jax.numpy as jnp
from jax import lax
from jax.experimental import pallas as pl
from jax.experimental.pallas import tpu as pltpu

N_DEV = 4
B, Sq, Hq, Hkv, Dh = 2, 256, 8, 2, 64
G = Hq // Hkv
SCALE = 0.125


def kernel(x, Wq, Wo, K_ext, V_ext):
    c = K_ext.shape[1]

    def body(x_ref, wq_ref, wo_ref, k_ref, v_ref, out_ref,
             kbuf, vbuf, k_send, k_recv, v_send, v_recv):
        my = lax.axis_index("i")

        barrier = pltpu.get_barrier_semaphore()
        for d in range(1, N_DEV):
            pl.semaphore_signal(barrier, inc=1, device_id=((my + d) % N_DEV,),
                                device_id_type=pl.DeviceIdType.MESH)
        pl.semaphore_wait(barrier, N_DEV - 1)

        for b in range(B):
            kb = k_ref[b].astype(jnp.bfloat16)
            vb = v_ref[b].astype(jnp.bfloat16)
            for kvh in range(Hkv):
                kbuf[0, b, kvh] = jnp.transpose(kb[:, kvh, :])
                vbuf[0, b, kvh] = vb[:, kvh, :]

        krd, vrd = {}, {}
        for d in (1, 3, 2):
            tgt = (my + d) % N_DEV
            krd[d] = pltpu.make_async_remote_copy(
                src_ref=kbuf.at[0], dst_ref=kbuf.at[N_DEV - d],
                send_sem=k_send.at[d - 1], recv_sem=k_recv.at[N_DEV - 1 - d],
                device_id=(tgt,), device_id_type=pl.DeviceIdType.MESH)
            vrd[d] = pltpu.make_async_remote_copy(
                src_ref=vbuf.at[0], dst_ref=vbuf.at[N_DEV - d],
                send_sem=v_send.at[d - 1], recv_sem=v_recv.at[N_DEV - 1 - d],
                device_id=(tgt,), device_id_type=pl.DeviceIdType.MESH)
            krd[d].start()
            vrd[d].start()

        qg = []
        for b in range(B):
            qb = jnp.dot(x_ref[b].astype(jnp.bfloat16),
                         wq_ref[...].astype(jnp.bfloat16),
                         preferred_element_type=jnp.float32)
            qbh = (qb * SCALE).astype(jnp.bfloat16)
            qg.append([
                jnp.concatenate(
                    [qbh[:, (kvh * G + g) * Dh:(kvh * G + g + 1) * Dh]
                     for g in range(G)], axis=0)
                for kvh in range(Hkv)
            ])

        l = [[jnp.zeros((G * Sq, 1), jnp.float32)
              for _ in range(Hkv)] for _ in range(B)]
        acc = [[jnp.zeros((G * Sq, Dh), jnp.float32)
                for _ in range(Hkv)] for _ in range(B)]

        def fold_slot(slot, k_wait=None, v_wait=None):
            if k_wait is not None:
                k_wait.wait_recv()
            ps = []
            for b in range(B):
                for kvh in range(Hkv):
                    kc = kbuf[slot, b, kvh]
                    s = jnp.dot(qg[b][kvh], kc,
                                preferred_element_type=jnp.float32)
                    p = jnp.exp(s)
                    l[b][kvh] = l[b][kvh] + jnp.sum(
                        p, axis=-1, keepdims=True)
                    ps.append(p.astype(jnp.bfloat16))
            if v_wait is not None:
                v_wait.wait_recv()
            for b in range(B):
                for kvh in range(Hkv):
                    vc = vbuf[slot, b, kvh]
                    acc[b][kvh] = acc[b][kvh] + jnp.dot(
                        ps[b * Hkv + kvh], vc,
                        preferred_element_type=jnp.float32)

        fold_slot(0)
        for slot, d in ((3, 1), (1, 3), (2, 2)):
            fold_slot(slot, k_wait=krd[d], v_wait=vrd[d])

        wo_b = wo_ref[...].astype(jnp.bfloat16)
        for b in range(B):
            heads = []
            for h in range(Hq):
                kvh, g = h // G, h % G
                o = (acc[b][kvh][g * Sq:(g + 1) * Sq, :]
                     / l[b][kvh][g * Sq:(g + 1) * Sq, :])
                heads.append(o)
            ob = jnp.concatenate(heads, axis=1)
            out_ref[b] = jnp.dot(ob.astype(jnp.bfloat16), wo_b,
                                 preferred_element_type=jnp.float32)

        for d in range(1, N_DEV):
            krd[d].wait_send()
            vrd[d].wait_send()

    return pl.pallas_call(
        body,
        out_shape=jax.ShapeDtypeStruct((B, Sq, Wo.shape[1]), jnp.float32),
        in_specs=[pl.BlockSpec(memory_space=pltpu.VMEM)] * 5,
        out_specs=pl.BlockSpec(memory_space=pltpu.VMEM),
        scratch_shapes=[
            pltpu.VMEM((N_DEV, B, Hkv, Dh, c), jnp.bfloat16),
            pltpu.VMEM((N_DEV, B, Hkv, c, Dh), jnp.bfloat16),
            pltpu.SemaphoreType.DMA((N_DEV - 1,)),
            pltpu.SemaphoreType.DMA((N_DEV - 1,)),
            pltpu.SemaphoreType.DMA((N_DEV - 1,)),
            pltpu.SemaphoreType.DMA((N_DEV - 1,)),
        ],
        compiler_params=pltpu.CompilerParams(collective_id=0),
    )(x, Wq, Wo, K_ext, V_ext)
